# baseline (speedup 1.0000x reference)
import functools

import jax
import jax.numpy as jnp
from jax import lax
from jax.experimental import pallas as pl
from jax.experimental.pallas import tpu as pltpu
import jax.experimental.pallas.tpu_sc as plsc

Q = 1024
K_DB = 100000
D_OUT = 384
TOP_K = 8


_BQ1 = 256
_NQB1 = Q // _BQ1


def _proj_body(t_ref, i_ref, gt_ref, w1_ref, b1_ref, w2_ref, b2_ref,
               w3_ref, b3_ref, proj_ref, qn_ref, loss_ref, acc_ref):
    q = pl.program_id(0)
    joint = jnp.concatenate([t_ref[...], i_ref[...]], axis=1)
    h = jnp.maximum(
        jnp.dot(joint, w1_ref[...], preferred_element_type=jnp.float32)
        + b1_ref[...], 0.0)
    h = jnp.maximum(
        jnp.dot(h, w2_ref[...], preferred_element_type=jnp.float32)
        + b2_ref[...], 0.0)
    p = jnp.dot(h, w3_ref[...], preferred_element_type=jnp.float32) + b3_ref[...]
    proj_ref[...] = p
    norm = jnp.sqrt(jnp.sum(p * p, axis=1, keepdims=True))
    qn_ref[...] = p / jnp.maximum(norm, 1e-12)
    gt = gt_ref[...]
    d = jnp.sum(p * gt, axis=1, keepdims=True)
    gnorm = jnp.sqrt(jnp.sum(gt * gt, axis=1, keepdims=True))
    cos = d / jnp.maximum(norm * gnorm, 1e-8)
    bsum = jnp.sum(1.0 - cos)

    @pl.when(q == 0)
    def _():
        acc_ref[0, 0] = 0.0

    acc_ref[0, 0] += bsum

    @pl.when(q == _NQB1 - 1)
    def _():
        loss_ref[...] = jnp.full((1, 1), acc_ref[0, 0] / float(Q), jnp.float32)


def _project(text_emb, image_emb, gt, W1, b1, W2, b2, W3, b3):
    return pl.pallas_call(
        _proj_body,
        grid=(_NQB1,),
        in_specs=[
            pl.BlockSpec((_BQ1, 768), lambda q: (q, 0)),
            pl.BlockSpec((_BQ1, 768), lambda q: (q, 0)),
            pl.BlockSpec((_BQ1, D_OUT), lambda q: (q, 0)),
            pl.BlockSpec((1536, 1024), lambda q: (0, 0)),
            pl.BlockSpec((1, 1024), lambda q: (0, 0)),
            pl.BlockSpec((1024, 512), lambda q: (0, 0)),
            pl.BlockSpec((1, 512), lambda q: (0, 0)),
            pl.BlockSpec((512, D_OUT), lambda q: (0, 0)),
            pl.BlockSpec((1, D_OUT), lambda q: (0, 0)),
        ],
        out_specs=[
            pl.BlockSpec((_BQ1, D_OUT), lambda q: (q, 0)),
            pl.BlockSpec((_BQ1, D_OUT), lambda q: (q, 0)),
            pl.BlockSpec((1, 1), lambda q: (0, 0)),
        ],
        out_shape=[
            jax.ShapeDtypeStruct((Q, D_OUT), jnp.float32),
            jax.ShapeDtypeStruct((Q, D_OUT), jnp.float32),
            jax.ShapeDtypeStruct((1, 1), jnp.float32),
        ],
        scratch_shapes=[pltpu.SMEM((1, 1), jnp.float32)],
    )(text_emb, image_emb, gt, W1, b1, W2, b2, W3, b3)


_BK = 1024
_NKB = -(-K_DB // _BK)
_NEG = float("-inf")
_BIG = 2.0e9


def _topk_fast_body(qn_ref, keys_ref, ti_ref, tie_ref, bs_ref, bi_ref,
                    acc_ref):
    k = pl.program_id(0)
    s = lax.dot_general(qn_ref[...], keys_ref[...],
                        dimension_numbers=(((1,), (1,)), ((), ())),
                        preferred_element_type=jnp.float32)
    iota = lax.broadcasted_iota(jnp.int32, (Q, _BK), 1).astype(jnp.float32)
    s = jnp.where(iota < (K_DB - k * _BK).astype(jnp.float32), s, _NEG)
    base = jnp.float32(k * _BK)

    @pl.when(k == 0)
    def _():
        bs_ref[...] = jnp.full((Q, TOP_K), _NEG, jnp.float32)
        bi_ref[...] = jnp.zeros((Q, TOP_K), jnp.float32)
        acc_ref[0, 0] = 0.0

    ms = [jnp.max(s, axis=1, keepdims=True)]
    for _ in range(TOP_K - 1):
        ms.append(jnp.max(jnp.where(s < ms[-1], s, _NEG), axis=1,
                          keepdims=True))
    ams = []
    for t in range(TOP_K):
        c = jnp.where(s == ms[t], iota, _BIG)
        ams.append(jnp.min(c, axis=1, keepdims=True) + base)
    cnt = jnp.sum(jnp.where(s >= ms[-1], 1.0, 0.0), axis=1, keepdims=True)
    acc_ref[0, 0] = jnp.maximum(acc_ref[0, 0], jnp.max(cnt))

    comb_s = jnp.concatenate([bs_ref[...]] + ms, axis=1)
    comb_i = jnp.concatenate([bi_ref[...]] + ams, axis=1)
    new_s, new_i = [], []
    for _ in range(TOP_K):
        m = jnp.max(comb_s, axis=1, keepdims=True)
        c = jnp.where(comb_s == m, comb_i, _BIG)
        am = jnp.min(c, axis=1, keepdims=True)
        comb_s = jnp.where(c == am, _NEG, comb_s)
        new_s.append(m)
        new_i.append(am)
    bs_ref[...] = jnp.concatenate(new_s, axis=1)
    bi_ref[...] = jnp.concatenate(new_i, axis=1)

    @pl.when(k == _NKB - 1)
    def _():
        ti_ref[...] = bi_ref[...].astype(jnp.int32)
        tie_ref[...] = jnp.full((1, 1), acc_ref[0, 0], jnp.float32)


def _topk_slow_body(qn_ref, keys_ref, ti_ref, bs_ref, bi_ref):
    k = pl.program_id(0)
    s = lax.dot_general(qn_ref[...], keys_ref[...],
                        dimension_numbers=(((1,), (1,)), ((), ())),
                        preferred_element_type=jnp.float32)
    iota = lax.broadcasted_iota(jnp.int32, (Q, _BK), 1).astype(jnp.float32)
    s = jnp.where(iota < (K_DB - k * _BK).astype(jnp.float32), s, _NEG)
    base = jnp.float32(k * _BK)

    @pl.when(k == 0)
    def _():
        bs_ref[...] = jnp.full((Q, TOP_K), _NEG, jnp.float32)
        bi_ref[...] = jnp.zeros((Q, TOP_K), jnp.float32)

    ms, ams = [], []
    for _ in range(TOP_K):
        m = jnp.max(s, axis=1, keepdims=True)
        c = jnp.where(s == m, iota, _BIG)
        am = jnp.min(c, axis=1, keepdims=True)
        s = jnp.where(c == am, _NEG, s)
        ms.append(m)
        ams.append(am + base)

    comb_s = jnp.concatenate([bs_ref[...]] + ms, axis=1)
    comb_i = jnp.concatenate([bi_ref[...]] + ams, axis=1)
    new_s, new_i = [], []
    for _ in range(TOP_K):
        m = jnp.max(comb_s, axis=1, keepdims=True)
        c = jnp.where(comb_s == m, comb_i, _BIG)
        am = jnp.min(c, axis=1, keepdims=True)
        comb_s = jnp.where(c == am, _NEG, comb_s)
        new_s.append(m)
        new_i.append(am)
    bs_ref[...] = jnp.concatenate(new_s, axis=1)
    bi_ref[...] = jnp.concatenate(new_i, axis=1)

    @pl.when(k == _NKB - 1)
    def _():
        ti_ref[...] = bi_ref[...].astype(jnp.int32)


def _topk(qn, keys):
    ti, tie = pl.pallas_call(
        _topk_fast_body,
        grid=(_NKB,),
        in_specs=[
            pl.BlockSpec((Q, D_OUT), lambda k: (0, 0)),
            pl.BlockSpec((_BK, D_OUT), lambda k: (k, 0)),
        ],
        out_specs=[
            pl.BlockSpec((Q, TOP_K), lambda k: (0, 0)),
            pl.BlockSpec((1, 1), lambda k: (0, 0)),
        ],
        out_shape=[
            jax.ShapeDtypeStruct((Q, TOP_K), jnp.int32),
            jax.ShapeDtypeStruct((1, 1), jnp.float32),
        ],
        scratch_shapes=[
            pltpu.VMEM((Q, TOP_K), jnp.float32),
            pltpu.VMEM((Q, TOP_K), jnp.float32),
            pltpu.SMEM((1, 1), jnp.float32),
        ],
    )(qn, keys)

    def _slow():
        return pl.pallas_call(
            _topk_slow_body,
            grid=(_NKB,),
            in_specs=[
                pl.BlockSpec((Q, D_OUT), lambda k: (0, 0)),
                pl.BlockSpec((_BK, D_OUT), lambda k: (k, 0)),
            ],
            out_specs=pl.BlockSpec((Q, TOP_K), lambda k: (0, 0)),
            out_shape=jax.ShapeDtypeStruct((Q, TOP_K), jnp.int32),
            scratch_shapes=[
                pltpu.VMEM((Q, TOP_K), jnp.float32),
                pltpu.VMEM((Q, TOP_K), jnp.float32),
            ],
        )(qn, keys)

    return lax.cond(tie[0, 0] > float(TOP_K), _slow, lambda: ti)


_NC = 2
_NS = 16
_NW = _NC * _NS
_ROWS = Q * TOP_K
_GCH = 128
_GROWS = _ROWS // _GCH


def _gather_body(keys_hbm, idx_hbm, out_hbm, idx_v, rows_v, sem):
    wid = lax.axis_index("s") * _NC + lax.axis_index("c")
    for j in range(_GROWS // _NW):
        r = wid * (_GROWS // _NW) + j
        pltpu.sync_copy(idx_hbm.at[r], idx_v)
        pltpu.async_copy(keys_hbm.at[idx_v], rows_v, sem).wait()
        pltpu.sync_copy(rows_v, out_hbm.at[pl.ds(r * _GCH, _GCH)])


def _gather(keys, topk_idx):
    mesh = plsc.VectorSubcoreMesh(core_axis_name="c", subcore_axis_name="s")
    g = functools.partial(
        pl.kernel,
        out_type=jax.ShapeDtypeStruct((_ROWS, D_OUT), jnp.float32),
        mesh=mesh,
        scratch_types=[
            pltpu.VMEM((_GCH,), jnp.int32),
            pltpu.VMEM((_GCH, D_OUT), jnp.float32),
            pltpu.SemaphoreType.DMA,
        ],
    )(_gather_body)
    return g(keys, topk_idx.reshape(_GROWS, _GCH))


def kernel(text_emb, image_emb, gt_retrievals_emb, W1, b1, W2, b2, W3, b3,
           keys):
    proj, qn, loss = _project(
        text_emb, image_emb, gt_retrievals_emb, W1,
        b1.reshape(1, -1), W2, b2.reshape(1, -1), W3, b3.reshape(1, -1))
    topk_idx = _topk(qn, keys)
    retrieved = _gather(keys, topk_idx).reshape(Q, TOP_K, D_OUT)
    return retrieved, proj, loss[0, 0]


# speedup vs baseline: 2.2144x; 2.2144x over previous
import functools

import jax
import jax.numpy as jnp
from jax import lax
from jax.experimental import pallas as pl
from jax.experimental.pallas import tpu as pltpu
import jax.experimental.pallas.tpu_sc as plsc

Q = 1024
K_DB = 100000
D_OUT = 384
TOP_K = 8


_BQ1 = 256
_NQB1 = Q // _BQ1


def _proj_body(t_ref, i_ref, gt_ref, w1_ref, b1_ref, w2_ref, b2_ref,
               w3_ref, b3_ref, proj_ref, qn_ref, loss_ref, acc_ref):
    q = pl.program_id(0)
    joint = jnp.concatenate([t_ref[...], i_ref[...]], axis=1)
    h = jnp.maximum(
        jnp.dot(joint, w1_ref[...], preferred_element_type=jnp.float32)
        + b1_ref[...], 0.0)
    h = jnp.maximum(
        jnp.dot(h, w2_ref[...], preferred_element_type=jnp.float32)
        + b2_ref[...], 0.0)
    p = jnp.dot(h, w3_ref[...], preferred_element_type=jnp.float32) + b3_ref[...]
    proj_ref[...] = p
    norm = jnp.sqrt(jnp.sum(p * p, axis=1, keepdims=True))
    qn_ref[...] = p / jnp.maximum(norm, 1e-12)
    gt = gt_ref[...]
    d = jnp.sum(p * gt, axis=1, keepdims=True)
    gnorm = jnp.sqrt(jnp.sum(gt * gt, axis=1, keepdims=True))
    cos = d / jnp.maximum(norm * gnorm, 1e-8)
    bsum = jnp.sum(1.0 - cos)

    @pl.when(q == 0)
    def _():
        acc_ref[0, 0] = 0.0

    acc_ref[0, 0] += bsum

    @pl.when(q == _NQB1 - 1)
    def _():
        loss_ref[...] = jnp.full((1, 1), acc_ref[0, 0] / float(Q), jnp.float32)


def _project(text_emb, image_emb, gt, W1, b1, W2, b2, W3, b3):
    return pl.pallas_call(
        _proj_body,
        grid=(_NQB1,),
        in_specs=[
            pl.BlockSpec((_BQ1, 768), lambda q: (q, 0)),
            pl.BlockSpec((_BQ1, 768), lambda q: (q, 0)),
            pl.BlockSpec((_BQ1, D_OUT), lambda q: (q, 0)),
            pl.BlockSpec((1536, 1024), lambda q: (0, 0)),
            pl.BlockSpec((1, 1024), lambda q: (0, 0)),
            pl.BlockSpec((1024, 512), lambda q: (0, 0)),
            pl.BlockSpec((1, 512), lambda q: (0, 0)),
            pl.BlockSpec((512, D_OUT), lambda q: (0, 0)),
            pl.BlockSpec((1, D_OUT), lambda q: (0, 0)),
        ],
        out_specs=[
            pl.BlockSpec((_BQ1, D_OUT), lambda q: (q, 0)),
            pl.BlockSpec((_BQ1, D_OUT), lambda q: (q, 0)),
            pl.BlockSpec((1, 1), lambda q: (0, 0)),
        ],
        out_shape=[
            jax.ShapeDtypeStruct((Q, D_OUT), jnp.float32),
            jax.ShapeDtypeStruct((Q, D_OUT), jnp.float32),
            jax.ShapeDtypeStruct((1, 1), jnp.float32),
        ],
        scratch_shapes=[pltpu.SMEM((1, 1), jnp.float32)],
    )(text_emb, image_emb, gt, W1, b1, W2, b2, W3, b3)


_BK = 2048
_NKB = -(-K_DB // _BK)
_NEG = float("-inf")
_BIG = 2.0e9


def _topk_body(qn_ref, keys_ref, ti_ref, bs_ref, bi_ref):
    k = pl.program_id(0)
    s = lax.dot_general(qn_ref[...], keys_ref[...],
                        dimension_numbers=(((1,), (1,)), ((), ())),
                        preferred_element_type=jnp.float32)
    iota = lax.broadcasted_iota(jnp.int32, (Q, _BK), 1).astype(jnp.float32)
    s = jnp.where(iota < (K_DB - k * _BK).astype(jnp.float32), s, _NEG)
    base = jnp.float32(k * _BK)

    @pl.when(k == 0)
    def _():
        bs_ref[...] = jnp.full((Q, TOP_K), _NEG, jnp.float32)
        bi_ref[...] = jnp.zeros((Q, TOP_K), jnp.float32)

    ms, ams = [], []
    for _ in range(TOP_K):
        m = jnp.max(s, axis=1, keepdims=True)
        c = jnp.where(s == m, iota, _BIG)
        am = jnp.min(c, axis=1, keepdims=True)
        s = jnp.where(c == am, _NEG, s)
        ms.append(m)
        ams.append(am + base)

    comb_s = jnp.concatenate([bs_ref[...]] + ms, axis=1)
    comb_i = jnp.concatenate([bi_ref[...]] + ams, axis=1)
    new_s, new_i = [], []
    for _ in range(TOP_K):
        m = jnp.max(comb_s, axis=1, keepdims=True)
        c = jnp.where(comb_s == m, comb_i, _BIG)
        am = jnp.min(c, axis=1, keepdims=True)
        comb_s = jnp.where(c == am, _NEG, comb_s)
        new_s.append(m)
        new_i.append(am)
    bs_ref[...] = jnp.concatenate(new_s, axis=1)
    bi_ref[...] = jnp.concatenate(new_i, axis=1)

    @pl.when(k == _NKB - 1)
    def _():
        ti_ref[...] = bi_ref[...].astype(jnp.int32)


def _topk(qn, keys):
    return pl.pallas_call(
        _topk_body,
        grid=(_NKB,),
        in_specs=[
            pl.BlockSpec((Q, D_OUT), lambda k: (0, 0)),
            pl.BlockSpec((_BK, D_OUT), lambda k: (k, 0)),
        ],
        out_specs=pl.BlockSpec((Q, TOP_K), lambda k: (0, 0)),
        out_shape=jax.ShapeDtypeStruct((Q, TOP_K), jnp.int32),
        scratch_shapes=[
            pltpu.VMEM((Q, TOP_K), jnp.float32),
            pltpu.VMEM((Q, TOP_K), jnp.float32),
        ],
    )(qn, keys)


_NC = 2
_NS = 16
_NW = _NC * _NS
_ROWS = Q * TOP_K
_GCH = 128
_GROWS = _ROWS // _GCH


def _gather_body(keys_hbm, idx_hbm, out_hbm, idx_v, rows_v, sem):
    wid = lax.axis_index("s") * _NC + lax.axis_index("c")
    for j in range(_GROWS // _NW):
        r = wid * (_GROWS // _NW) + j
        pltpu.sync_copy(idx_hbm.at[r], idx_v)
        pltpu.async_copy(keys_hbm.at[idx_v], rows_v, sem).wait()
        pltpu.sync_copy(rows_v, out_hbm.at[pl.ds(r * _GCH, _GCH)])


def _gather(keys, topk_idx):
    mesh = plsc.VectorSubcoreMesh(core_axis_name="c", subcore_axis_name="s")
    g = functools.partial(
        pl.kernel,
        out_type=jax.ShapeDtypeStruct((_ROWS, D_OUT), jnp.float32),
        mesh=mesh,
        scratch_types=[
            pltpu.VMEM((_GCH,), jnp.int32),
            pltpu.VMEM((_GCH, D_OUT), jnp.float32),
            pltpu.SemaphoreType.DMA,
        ],
    )(_gather_body)
    return g(keys, topk_idx.reshape(_GROWS, _GCH))


def kernel(text_emb, image_emb, gt_retrievals_emb, W1, b1, W2, b2, W3, b3,
           keys):
    proj, qn, loss = _project(
        text_emb, image_emb, gt_retrievals_emb, W1,
        b1.reshape(1, -1), W2, b2.reshape(1, -1), W3, b3.reshape(1, -1))
    topk_idx = _topk(qn, keys)
    retrieved = _gather(keys, topk_idx).reshape(Q, TOP_K, D_OUT)
    return retrieved, proj, loss[0, 0]


# speedup vs baseline: 2.2650x; 1.0228x over previous
import functools

import jax
import jax.numpy as jnp
from jax import lax
from jax.experimental import pallas as pl
from jax.experimental.pallas import tpu as pltpu
import jax.experimental.pallas.tpu_sc as plsc

Q = 1024
K_DB = 100000
D_OUT = 384
TOP_K = 8


_BQ1 = 256
_NQB1 = Q // _BQ1


def _proj_body(t_ref, i_ref, gt_ref, w1_ref, b1_ref, w2_ref, b2_ref,
               w3_ref, b3_ref, proj_ref, qn_ref, loss_ref, acc_ref):
    q = pl.program_id(0)
    joint = jnp.concatenate([t_ref[...], i_ref[...]], axis=1)
    h = jnp.maximum(
        jnp.dot(joint, w1_ref[...], preferred_element_type=jnp.float32)
        + b1_ref[...], 0.0)
    h = jnp.maximum(
        jnp.dot(h, w2_ref[...], preferred_element_type=jnp.float32)
        + b2_ref[...], 0.0)
    p = jnp.dot(h, w3_ref[...], preferred_element_type=jnp.float32) + b3_ref[...]
    proj_ref[...] = p
    norm = jnp.sqrt(jnp.sum(p * p, axis=1, keepdims=True))
    qn_ref[...] = p / jnp.maximum(norm, 1e-12)
    gt = gt_ref[...]
    d = jnp.sum(p * gt, axis=1, keepdims=True)
    gnorm = jnp.sqrt(jnp.sum(gt * gt, axis=1, keepdims=True))
    cos = d / jnp.maximum(norm * gnorm, 1e-8)
    bsum = jnp.sum(1.0 - cos)

    @pl.when(q == 0)
    def _():
        acc_ref[0, 0] = 0.0

    acc_ref[0, 0] += bsum

    @pl.when(q == _NQB1 - 1)
    def _():
        loss_ref[...] = jnp.full((1, 1), acc_ref[0, 0] / float(Q), jnp.float32)


def _project(text_emb, image_emb, gt, W1, b1, W2, b2, W3, b3):
    return pl.pallas_call(
        _proj_body,
        grid=(_NQB1,),
        in_specs=[
            pl.BlockSpec((_BQ1, 768), lambda q: (q, 0)),
            pl.BlockSpec((_BQ1, 768), lambda q: (q, 0)),
            pl.BlockSpec((_BQ1, D_OUT), lambda q: (q, 0)),
            pl.BlockSpec((1536, 1024), lambda q: (0, 0)),
            pl.BlockSpec((1, 1024), lambda q: (0, 0)),
            pl.BlockSpec((1024, 512), lambda q: (0, 0)),
            pl.BlockSpec((1, 512), lambda q: (0, 0)),
            pl.BlockSpec((512, D_OUT), lambda q: (0, 0)),
            pl.BlockSpec((1, D_OUT), lambda q: (0, 0)),
        ],
        out_specs=[
            pl.BlockSpec((_BQ1, D_OUT), lambda q: (q, 0)),
            pl.BlockSpec((_BQ1, D_OUT), lambda q: (q, 0)),
            pl.BlockSpec((1, 1), lambda q: (0, 0)),
        ],
        out_shape=[
            jax.ShapeDtypeStruct((Q, D_OUT), jnp.float32),
            jax.ShapeDtypeStruct((Q, D_OUT), jnp.float32),
            jax.ShapeDtypeStruct((1, 1), jnp.float32),
        ],
        scratch_shapes=[pltpu.SMEM((1, 1), jnp.float32)],
    )(text_emb, image_emb, gt, W1, b1, W2, b2, W3, b3)


_BK = 4096
_NKB = -(-K_DB // _BK)
_NEG = float("-inf")
_BIG = 2.0e9


def _topk_body(qn_ref, keys_ref, ti_ref, bs_ref, bi_ref):
    k = pl.program_id(0)
    s = lax.dot_general(qn_ref[...], keys_ref[...],
                        dimension_numbers=(((1,), (1,)), ((), ())),
                        preferred_element_type=jnp.float32)
    iota = lax.broadcasted_iota(jnp.int32, (Q, _BK), 1).astype(jnp.float32)
    s = jnp.where(iota < (K_DB - k * _BK).astype(jnp.float32), s, _NEG)
    base = jnp.float32(k * _BK)

    @pl.when(k == 0)
    def _():
        bs_ref[...] = jnp.full((Q, TOP_K), _NEG, jnp.float32)
        bi_ref[...] = jnp.zeros((Q, TOP_K), jnp.float32)

    ms, ams = [], []
    for _ in range(TOP_K):
        m = jnp.max(s, axis=1, keepdims=True)
        c = jnp.where(s == m, iota, _BIG)
        am = jnp.min(c, axis=1, keepdims=True)
        s = jnp.where(c == am, _NEG, s)
        ms.append(m)
        ams.append(am + base)

    comb_s = jnp.concatenate([bs_ref[...]] + ms, axis=1)
    comb_i = jnp.concatenate([bi_ref[...]] + ams, axis=1)
    new_s, new_i = [], []
    for _ in range(TOP_K):
        m = jnp.max(comb_s, axis=1, keepdims=True)
        c = jnp.where(comb_s == m, comb_i, _BIG)
        am = jnp.min(c, axis=1, keepdims=True)
        comb_s = jnp.where(c == am, _NEG, comb_s)
        new_s.append(m)
        new_i.append(am)
    bs_ref[...] = jnp.concatenate(new_s, axis=1)
    bi_ref[...] = jnp.concatenate(new_i, axis=1)

    @pl.when(k == _NKB - 1)
    def _():
        ti_ref[...] = bi_ref[...].astype(jnp.int32)


def _topk(qn, keys):
    return pl.pallas_call(
        _topk_body,
        grid=(_NKB,),
        in_specs=[
            pl.BlockSpec((Q, D_OUT), lambda k: (0, 0)),
            pl.BlockSpec((_BK, D_OUT), lambda k: (k, 0)),
        ],
        out_specs=pl.BlockSpec((Q, TOP_K), lambda k: (0, 0)),
        out_shape=jax.ShapeDtypeStruct((Q, TOP_K), jnp.int32),
        scratch_shapes=[
            pltpu.VMEM((Q, TOP_K), jnp.float32),
            pltpu.VMEM((Q, TOP_K), jnp.float32),
        ],
    )(qn, keys)


_NC = 2
_NS = 16
_NW = _NC * _NS
_ROWS = Q * TOP_K
_GCH = 128
_GROWS = _ROWS // _GCH


def _gather_body(keys_hbm, idx_hbm, out_hbm, idx_v, rows_v, sem):
    wid = lax.axis_index("s") * _NC + lax.axis_index("c")
    for j in range(_GROWS // _NW):
        r = wid * (_GROWS // _NW) + j
        pltpu.sync_copy(idx_hbm.at[r], idx_v)
        pltpu.async_copy(keys_hbm.at[idx_v], rows_v, sem).wait()
        pltpu.sync_copy(rows_v, out_hbm.at[pl.ds(r * _GCH, _GCH)])


def _gather(keys, topk_idx):
    mesh = plsc.VectorSubcoreMesh(core_axis_name="c", subcore_axis_name="s")
    g = functools.partial(
        pl.kernel,
        out_type=jax.ShapeDtypeStruct((_ROWS, D_OUT), jnp.float32),
        mesh=mesh,
        scratch_types=[
            pltpu.VMEM((_GCH,), jnp.int32),
            pltpu.VMEM((_GCH, D_OUT), jnp.float32),
            pltpu.SemaphoreType.DMA,
        ],
    )(_gather_body)
    return g(keys, topk_idx.reshape(_GROWS, _GCH))


def kernel(text_emb, image_emb, gt_retrievals_emb, W1, b1, W2, b2, W3, b3,
           keys):
    proj, qn, loss = _project(
        text_emb, image_emb, gt_retrievals_emb, W1,
        b1.reshape(1, -1), W2, b2.reshape(1, -1), W3, b3.reshape(1, -1))
    topk_idx = _topk(qn, keys)
    retrieved = _gather(keys, topk_idx).reshape(Q, TOP_K, D_OUT)
    return retrieved, proj, loss[0, 0]
